# baseline (speedup 1.0000x reference)
import jax
import jax.numpy as jnp
from jax import lax
from jax.experimental import pallas as pl
from jax.experimental.pallas import tpu as pltpu
from jax.experimental.pallas import tpu_sc as plsc

B, L, D = 4096, 200, 32
NBINS = 10
NC, NS, LANES = 2, 16, 16
NW = NC * NS
ROWS_PER_W = B // NW
LPAD = 208
NGROUPS = LPAD // LANES
ACC = NBINS * D
CHUNKS = ((0, 128), (128, 80))


def _sc_embedder(t_hbm, idx_hbm, wx_hbm, ww_hbm, out_hbm,
                 t_v, idx_v, ww_v, emb_v, acc_v, cnt_v, sem):
    wid = lax.axis_index("s") * NC + lax.axis_index("c")
    b0 = wid * ROWS_PER_W

    def row_body(i, carry):
        b = b0 + i
        idx_v[pl.ds(LPAD - LANES, LANES)] = jnp.zeros((LANES,), jnp.int32)
        pltpu.sync_copy(t_hbm.at[pl.ds(b * L, L)], t_v.at[pl.ds(0, L)])
        pltpu.sync_copy(idx_hbm.at[pl.ds(b * L, L)], idx_v.at[pl.ds(0, L)])
        handles = []
        for off, sz in CHUNKS:
            handles.append(pltpu.async_copy(
                wx_hbm.at[idx_v.at[pl.ds(off, sz)]],
                emb_v.at[pl.ds(off, sz)], sem))
            handles.append(pltpu.async_copy(
                ww_hbm.at[idx_v.at[pl.ds(off, sz)]],
                ww_v.at[pl.ds(off, sz)], sem))
        zf = jnp.zeros((LANES,), jnp.float32)
        for r in range(ACC // LANES):
            acc_v[pl.ds(r * LANES, LANES)] = zf
        cnt_v[...] = zf
        for h in handles:
            h.wait()

        ones = jnp.ones((LANES,), jnp.float32)
        for g in range(NGROUPS):
            toks = lax.iota(jnp.int32, LANES) + g * LANES
            t16 = t_v[pl.ds(g * LANES, LANES)]
            bins = t16.astype(jnp.int32)
            mask = t16 < 10.0
            if (g + 1) * LANES > L:
                mask = mask & (toks < L)
            w16 = jnp.exp(ww_v[pl.ds(g * LANES, LANES)])
            plsc.addupdate_scatter(cnt_v, [bins + 1], ones, mask=mask)
            bins32 = bins * D
            for d in range(D):
                dfull = jnp.full((LANES,), d, jnp.int32)
                vals = plsc.load_gather(emb_v, [toks, dfull])
                plsc.addupdate_scatter(acc_v, [bins32 + d], vals * w16,
                                       mask=mask)

        for bn in range(NBINS):
            c = plsc.load_gather(cnt_v, [jnp.full((LANES,), bn + 1, jnp.int32)])
            denom = c + 1e-6
            for h in range(2):
                off = bn * D + h * LANES
                acc_v[pl.ds(off, LANES)] = acc_v[pl.ds(off, LANES)] / denom
        pltpu.sync_copy(acc_v.at[pl.ds(0, ACC)],
                        out_hbm.at[pl.ds(b * ACC, ACC)])
        return carry

    lax.fori_loop(0, ROWS_PER_W, row_body, 0)


def _build():
    mesh = plsc.VectorSubcoreMesh(core_axis_name="c", subcore_axis_name="s")
    return pl.kernel(
        _sc_embedder,
        out_type=jax.ShapeDtypeStruct((B * NBINS * D,), jnp.float32),
        mesh=mesh,
        scratch_types=[
            pltpu.VMEM((LPAD,), jnp.float32),
            pltpu.VMEM((LPAD,), jnp.int32),
            pltpu.VMEM((LPAD,), jnp.float32),
            pltpu.VMEM((LPAD, D), jnp.float32),
            pltpu.VMEM((ACC,), jnp.float32),
            pltpu.VMEM((LANES,), jnp.float32),
            pltpu.SemaphoreType.DMA,
        ],
        compiler_params=pltpu.CompilerParams(needs_layout_passes=False,
                                             use_tc_tiling_on_sc=False),
    )


def kernel(X, Wx, Ww):
    T = X[:, :, 0].reshape(-1)
    idx = X[:, :, 1].astype(jnp.int32).reshape(-1)
    fn = _build()
    out = fn(T, idx, Wx, Ww.reshape(-1))
    return out.reshape(B, NBINS, D)


# speedup vs baseline: 1.0748x; 1.0748x over previous
import jax
import jax.numpy as jnp
from jax import lax
from jax.experimental import pallas as pl
from jax.experimental.pallas import tpu as pltpu
from jax.experimental.pallas import tpu_sc as plsc

B, L, D = 4096, 200, 32
NBINS = 10
NC, NS, LANES = 2, 16, 16
NW = NC * NS
ROWS_PER_W = B // NW
BLK = ROWS_PER_W * L
LPAD = 208
NGROUPS = LPAD // LANES
ACC = NBINS * D
CHUNKS = ((0, 128), (128, 80))


def _sc_embedder(t_hbm, idx_hbm, wx_hbm, ww_hbm, out_hbm,
                 t_all, idx_all, ww_v, emb_v, acc_v, cnt_v, sem_g, sem_o):
    wid = lax.axis_index("s") * NC + lax.axis_index("c")
    base = wid * BLK

    def gather_descs(i, buf):
        ds = []
        for off, sz in CHUNKS:
            idxsl = idx_all.at[pl.ds(i * L + off, sz)]
            ds.append(pltpu.make_async_copy(
                wx_hbm.at[idxsl], emb_v[buf].at[pl.ds(off, sz)], sem_g[buf]))
            ds.append(pltpu.make_async_copy(
                ww_hbm.at[idxsl], ww_v[buf].at[pl.ds(off, sz)], sem_g[buf]))
        return ds

    def fire(i, buf):
        for d in gather_descs(i, buf):
            d.start()

    def wait(i, buf):
        for d in gather_descs(i, buf):
            d.wait()

    def out_desc(i, buf):
        return pltpu.make_async_copy(
            acc_v[buf].at[pl.ds(0, ACC)],
            out_hbm.at[pl.ds((wid * ROWS_PER_W + i) * ACC, ACC)], sem_o[buf])

    def compute(i, buf):
        acc = acc_v[buf]
        cnt = cnt_v[buf]
        zf = jnp.zeros((LANES,), jnp.float32)
        for r in range(ACC // LANES):
            acc[pl.ds(r * LANES, LANES)] = zf
        cnt[...] = zf
        wait(i, buf)
        ones = jnp.ones((LANES,), jnp.float32)
        tb = i * L
        for g in range(NGROUPS):
            toks = lax.iota(jnp.int32, LANES) + g * LANES
            t16 = t_all[pl.ds(tb + g * LANES, LANES)]
            bins = t16.astype(jnp.int32)
            mask = t16 < 10.0
            if (g + 1) * LANES > L:
                mask = mask & (toks < L)
            w16 = jnp.exp(ww_v[buf][pl.ds(g * LANES, LANES)])
            plsc.addupdate_scatter(cnt, [bins + 1], ones, mask=mask)
            bins32 = bins * D
            for d in range(D):
                dfull = jnp.full((LANES,), d, jnp.int32)
                vals = plsc.load_gather(emb_v[buf], [toks, dfull])
                plsc.addupdate_scatter(acc, [bins32 + d], vals * w16,
                                       mask=mask)
        for bn in range(NBINS):
            c = plsc.load_gather(cnt, [jnp.full((LANES,), bn + 1, jnp.int32)])
            denom = c + 1e-6
            for h in range(2):
                off = bn * D + h * LANES
                acc[pl.ds(off, LANES)] = acc[pl.ds(off, LANES)] / denom
        out_desc(i, buf).start()

    idx_all[pl.ds(BLK - 8, LANES)] = jnp.zeros((LANES,), jnp.int32)
    pltpu.sync_copy(t_hbm.at[pl.ds(base, BLK)], t_all.at[pl.ds(0, BLK)])
    pltpu.sync_copy(idx_hbm.at[pl.ds(base, BLK)], idx_all.at[pl.ds(0, BLK)])

    fire(0, 0)

    def pair_body(p, carry):
        i0 = 2 * p
        i1 = i0 + 1
        fire(i1, 1)

        @pl.when(p > 0)
        def _():
            out_desc(i0 - 2, 0).wait()
        compute(i0, 0)

        @pl.when(p < ROWS_PER_W // 2 - 1)
        def _():
            fire(i0 + 2, 0)

        @pl.when(p > 0)
        def _():
            out_desc(i1 - 2, 1).wait()
        compute(i1, 1)
        return carry

    lax.fori_loop(0, ROWS_PER_W // 2, pair_body, 0)
    out_desc(ROWS_PER_W - 2, 0).wait()
    out_desc(ROWS_PER_W - 1, 1).wait()


def _build():
    mesh = plsc.VectorSubcoreMesh(core_axis_name="c", subcore_axis_name="s")
    return pl.kernel(
        _sc_embedder,
        out_type=jax.ShapeDtypeStruct((B * NBINS * D,), jnp.float32),
        mesh=mesh,
        scratch_types=[
            pltpu.VMEM((BLK + 8,), jnp.float32),
            pltpu.VMEM((BLK + 8,), jnp.int32),
            [pltpu.VMEM((LPAD,), jnp.float32)] * 2,
            [pltpu.VMEM((LPAD, D), jnp.float32)] * 2,
            [pltpu.VMEM((ACC,), jnp.float32)] * 2,
            [pltpu.VMEM((LANES,), jnp.float32)] * 2,
            [pltpu.SemaphoreType.DMA] * 2,
            [pltpu.SemaphoreType.DMA] * 2,
        ],
        compiler_params=pltpu.CompilerParams(needs_layout_passes=False,
                                             use_tc_tiling_on_sc=False),
    )


def kernel(X, Wx, Ww):
    T = X[:, :, 0].reshape(-1)
    idx = X[:, :, 1].astype(jnp.int32).reshape(-1)
    fn = _build()
    out = fn(T, idx, Wx, Ww.reshape(-1))
    return out.reshape(B, NBINS, D)


# speedup vs baseline: 1.4205x; 1.3217x over previous
import jax
import jax.numpy as jnp
from jax import lax
from jax.experimental import pallas as pl
from jax.experimental.pallas import tpu as pltpu
from jax.experimental.pallas import tpu_sc as plsc

B, L, D = 4096, 200, 32
NBINS = 10
NC, NS, LANES = 2, 16, 16
NW = NC * NS
ROWS_PER_W = B // NW
BLK = ROWS_PER_W * L
LPAD = 208
NGROUPS = LPAD // LANES
ACC = NBINS * D
CHUNKS = ((0, 128), (128, 80))


def _sc_embedder(t_hbm, idx_hbm, wx_hbm, ww_hbm, out_hbm,
                 t_all, idx_all, ww_v, emb_v, acc_v, cnt_v, sem_g, sem_o):
    wid = lax.axis_index("s") * NC + lax.axis_index("c")
    base = wid * BLK

    def gather_descs(i, buf):
        ds = []
        for off, sz in CHUNKS:
            idxsl = idx_all.at[pl.ds(i * L + off, sz)]
            ds.append(pltpu.make_async_copy(
                wx_hbm.at[idxsl], emb_v[buf].at[pl.ds(off, sz)], sem_g[buf]))
            ds.append(pltpu.make_async_copy(
                ww_hbm.at[idxsl], ww_v[buf].at[pl.ds(off, sz)], sem_g[buf]))
        return ds

    def fire(i, buf):
        for d in gather_descs(i, buf):
            d.start()

    def wait(i, buf):
        for d in gather_descs(i, buf):
            d.wait()

    def out_desc(i, buf):
        return pltpu.make_async_copy(
            acc_v[buf].at[pl.ds(0, ACC)],
            out_hbm.at[pl.ds((wid * ROWS_PER_W + i) * ACC, ACC)], sem_o[buf])

    def compute(i, buf):
        acc = acc_v[buf]
        cnt = cnt_v[buf]
        zf = jnp.zeros((LANES,), jnp.float32)
        for r in range(ACC // LANES):
            acc[pl.ds(r * LANES, LANES)] = zf
        cnt[...] = zf
        wait(i, buf)
        ones = jnp.ones((LANES,), jnp.float32)
        tb = i * L
        for g in range(NGROUPS):
            toks = lax.iota(jnp.int32, LANES) + g * LANES
            t16 = t_all[pl.ds(tb + g * LANES, LANES)]
            bins = t16.astype(jnp.int32)
            mask = t16 < 10.0
            if (g + 1) * LANES > L:
                mask = mask & (toks < L)
            w16 = jnp.exp(ww_v[buf][pl.ds(g * LANES, LANES)])
            plsc.addupdate_scatter(cnt, [bins + 1], ones, mask=mask)
            bins32 = bins * D
            @plsc.parallel_loop(0, D, unroll=8)
            def _dim_body(d):
                dfull = jnp.full((LANES,), d, jnp.int32)
                vals = plsc.load_gather(emb_v[buf], [toks, dfull])
                plsc.addupdate_scatter(acc, [bins32 + dfull], vals * w16,
                                       mask=mask)
        for bn in range(NBINS):
            c = plsc.load_gather(cnt, [jnp.full((LANES,), bn + 1, jnp.int32)])
            denom = c + 1e-6
            for h in range(2):
                off = bn * D + h * LANES
                acc[pl.ds(off, LANES)] = acc[pl.ds(off, LANES)] / denom
        out_desc(i, buf).start()

    idx_all[pl.ds(BLK - 8, LANES)] = jnp.zeros((LANES,), jnp.int32)
    pltpu.sync_copy(t_hbm.at[pl.ds(base, BLK)], t_all.at[pl.ds(0, BLK)])
    pltpu.sync_copy(idx_hbm.at[pl.ds(base, BLK)], idx_all.at[pl.ds(0, BLK)])

    fire(0, 0)

    def pair_body(p, carry):
        i0 = 2 * p
        i1 = i0 + 1
        fire(i1, 1)

        @pl.when(p > 0)
        def _():
            out_desc(i0 - 2, 0).wait()
        compute(i0, 0)

        @pl.when(p < ROWS_PER_W // 2 - 1)
        def _():
            fire(i0 + 2, 0)

        @pl.when(p > 0)
        def _():
            out_desc(i1 - 2, 1).wait()
        compute(i1, 1)
        return carry

    lax.fori_loop(0, ROWS_PER_W // 2, pair_body, 0)
    out_desc(ROWS_PER_W - 2, 0).wait()
    out_desc(ROWS_PER_W - 1, 1).wait()


def _build():
    mesh = plsc.VectorSubcoreMesh(core_axis_name="c", subcore_axis_name="s")
    return pl.kernel(
        _sc_embedder,
        out_type=jax.ShapeDtypeStruct((B * NBINS * D,), jnp.float32),
        mesh=mesh,
        scratch_types=[
            pltpu.VMEM((BLK + 8,), jnp.float32),
            pltpu.VMEM((BLK + 8,), jnp.int32),
            [pltpu.VMEM((LPAD,), jnp.float32)] * 2,
            [pltpu.VMEM((LPAD, D), jnp.float32)] * 2,
            [pltpu.VMEM((ACC,), jnp.float32)] * 2,
            [pltpu.VMEM((LANES,), jnp.float32)] * 2,
            [pltpu.SemaphoreType.DMA] * 2,
            [pltpu.SemaphoreType.DMA] * 2,
        ],
        compiler_params=pltpu.CompilerParams(needs_layout_passes=False,
                                             use_tc_tiling_on_sc=False),
    )


def kernel(X, Wx, Ww):
    T = X[:, :, 0].reshape(-1)
    idx = X[:, :, 1].astype(jnp.int32).reshape(-1)
    fn = _build()
    out = fn(T, idx, Wx, Ww.reshape(-1))
    return out.reshape(B, NBINS, D)
